# baseline (speedup 1.0000x reference)
import math

import jax
import jax.numpy as jnp
from jax.experimental import pallas as pl
from jax.experimental.pallas import tpu as pltpu

N = 8192
D_MAIN = 128
D_MOD = 128
D_H = 16
D_QK = 8
D_AUG = 16
D_OUT = 128
TQ = 1024
GRID = N // TQ


def _fused_kernel(x_main_ref, x_mod_ref, w_e1_ref, w_e2_ref, w_q_ref,
                  w_k_ref, w_v_ref, out_ref, h_s, k_s, v_s, kmax_s):
    i = pl.program_id(0)
    col = jax.lax.broadcasted_iota(jnp.int32, (1, D_AUG), 1)
    e_aug = jnp.where(col == D_QK, 1.0, 0.0).astype(jnp.float32)

    @pl.when(i == 0)
    def _prologue():
        w1a = w_e1_ref[0:D_MAIN, :]
        w1b = w_e1_ref[D_MAIN:D_MAIN + D_MOD, :]
        h1 = jnp.maximum(
            jnp.dot(x_main_ref[...], w1a, preferred_element_type=jnp.float32)
            + jnp.dot(x_mod_ref[...], w1b, preferred_element_type=jnp.float32),
            0.0)
        h = jnp.maximum(
            jnp.dot(h1, w_e2_ref[...], preferred_element_type=jnp.float32), 0.0)
        h_s[...] = h
        k = jnp.dot(h, w_k_ref[...], preferred_element_type=jnp.float32)
        k_s[...] = k - e_aug
        kmax_s[0, 0] = jnp.sqrt(jnp.max(jnp.sum(k * k, axis=1)))
        vcol = jax.lax.broadcasted_iota(jnp.int32, (1, 2 * D_OUT), 1)
        e_vcol = jnp.where(vcol == D_OUT, 1.0, 0.0).astype(jnp.float32)
        v_s[...] = (jnp.dot(h, w_v_ref[...],
                            preferred_element_type=jnp.float32)
                    + e_vcol).astype(jnp.bfloat16)

    hq = h_s[pl.ds(i * TQ, TQ), :]
    q = jnp.dot(hq, w_q_ref[...], preferred_element_type=jnp.float32)
    b = jnp.sqrt(jnp.sum(q * q, axis=1, keepdims=True)) * kmax_s[0, 0]
    q_aug = q + b * e_aug
    scores = jax.lax.dot_general(
        q_aug, k_s[...], (((1,), (1,)), ((), ())),
        preferred_element_type=jnp.float32)
    e = jnp.exp(scores)
    o = jnp.dot(e.astype(jnp.bfloat16), v_s[...],
                preferred_element_type=jnp.float32)
    out_ref[...] = o[:, 0:D_OUT] / o[:, D_OUT:D_OUT + 1]


def kernel(x_main, x_mod, xyz, W_E1, W_E2, W_Q, W_K, W_V):
    del xyz
    scale = 1.0 / math.sqrt(D_QK)
    pad = jnp.zeros((D_H, D_AUG - D_QK), jnp.float32)
    w_q_aug = jnp.concatenate([W_Q * scale, pad], axis=1)
    w_k_aug = jnp.concatenate([W_K, pad], axis=1)
    w_v_aug = jnp.concatenate([W_V, jnp.zeros((D_H, D_OUT), jnp.float32)], axis=1)
    full = lambda s: pl.BlockSpec(s, lambda i: (0, 0))
    return pl.pallas_call(
        _fused_kernel,
        grid=(GRID,),
        in_specs=[
            full((N, D_MAIN)),
            full((N, D_MOD)),
            full((D_MAIN + D_MOD, D_H)),
            full((D_H, D_H)),
            full((D_H, D_AUG)),
            full((D_H, D_AUG)),
            full((D_H, 2 * D_OUT)),
        ],
        out_specs=pl.BlockSpec((TQ, D_OUT), lambda i: (i, 0)),
        out_shape=jax.ShapeDtypeStruct((N, D_OUT), jnp.float32),
        scratch_shapes=[
            pltpu.VMEM((N, D_H), jnp.float32),
            pltpu.VMEM((N, D_AUG), jnp.float32),
            pltpu.VMEM((N, 2 * D_OUT), jnp.bfloat16),
            pltpu.SMEM((1, 1), jnp.float32),
        ],
    )(x_main, x_mod, W_E1, W_E2, w_q_aug, w_k_aug, w_v_aug)


# speedup vs baseline: 1.0370x; 1.0370x over previous
import math

import jax
import jax.numpy as jnp
from jax.experimental import pallas as pl
from jax.experimental.pallas import tpu as pltpu

N = 8192
D_MAIN = 128
D_MOD = 128
D_H = 16
D_QK = 8
D_AUG = 16
D_OUT = 128
TQ = 1024
GRID = N // TQ


def _fused_kernel(x_main_ref, x_mod_ref, w_e1_ref, w_e2_ref, w_q_ref,
                  w_k_ref, w_v_ref, out_ref, h_s, k_s, v_s, kmax_s):
    i = pl.program_id(0)
    col = jax.lax.broadcasted_iota(jnp.int32, (1, D_AUG), 1)
    e_aug = jnp.where(col == D_QK, 1.0, 0.0).astype(jnp.float32)

    @pl.when(i == 0)
    def _prologue():
        w1a = w_e1_ref[0:D_MAIN, :]
        w1b = w_e1_ref[D_MAIN:D_MAIN + D_MOD, :]
        h1 = jnp.maximum(
            jnp.dot(x_main_ref[...], w1a, preferred_element_type=jnp.float32)
            + jnp.dot(x_mod_ref[...], w1b, preferred_element_type=jnp.float32),
            0.0)
        h = jnp.maximum(
            jnp.dot(h1, w_e2_ref[...], preferred_element_type=jnp.float32), 0.0)
        h_s[...] = h
        k = jnp.dot(h, w_k_ref[...], preferred_element_type=jnp.float32)
        k_s[...] = k - e_aug
        kmax_s[0, 0] = jnp.sqrt(jnp.max(jnp.sum(k * k, axis=1)))
        v_s[...] = jnp.dot(h, w_v_ref[...],
                           preferred_element_type=jnp.float32).astype(jnp.bfloat16)

    hq = h_s[pl.ds(i * TQ, TQ), :]
    q = jnp.dot(hq, w_q_ref[...], preferred_element_type=jnp.float32)
    b = jnp.sqrt(jnp.sum(q * q, axis=1, keepdims=True)) * kmax_s[0, 0]
    q_aug = q + b * e_aug
    scores = jax.lax.dot_general(
        q_aug, k_s[...], (((1,), (1,)), ((), ())),
        preferred_element_type=jnp.float32)
    e = jnp.exp2(scores)
    denom = jnp.sum(e, axis=1, keepdims=True)
    o = jnp.dot(e.astype(jnp.bfloat16), v_s[...],
                preferred_element_type=jnp.float32)
    out_ref[...] = o / denom


def kernel(x_main, x_mod, xyz, W_E1, W_E2, W_Q, W_K, W_V):
    del xyz
    scale = math.log2(math.e) / math.sqrt(D_QK)
    pad = jnp.zeros((D_H, D_AUG - D_QK), jnp.float32)
    w_q_aug = jnp.concatenate([W_Q * scale, pad], axis=1)
    w_k_aug = jnp.concatenate([W_K, pad], axis=1)
    full = lambda s: pl.BlockSpec(s, lambda i: (0, 0))
    return pl.pallas_call(
        _fused_kernel,
        grid=(GRID,),
        in_specs=[
            full((N, D_MAIN)),
            full((N, D_MOD)),
            full((D_MAIN + D_MOD, D_H)),
            full((D_H, D_H)),
            full((D_H, D_AUG)),
            full((D_H, D_AUG)),
            full((D_H, D_OUT)),
        ],
        out_specs=pl.BlockSpec((TQ, D_OUT), lambda i: (i, 0)),
        out_shape=jax.ShapeDtypeStruct((N, D_OUT), jnp.float32),
        scratch_shapes=[
            pltpu.VMEM((N, D_H), jnp.float32),
            pltpu.VMEM((N, D_AUG), jnp.float32),
            pltpu.VMEM((N, D_OUT), jnp.bfloat16),
            pltpu.SMEM((1, 1), jnp.float32),
        ],
    )(x_main, x_mod, W_E1, W_E2, w_q_aug, w_k_aug, W_V)
